# baseline (speedup 1.0000x reference)
import functools
import math

import jax
import jax.numpy as jnp
import numpy as np
from jax import lax
from jax.experimental import pallas as pl
from jax.experimental.pallas import tpu as pltpu
from jax.experimental.pallas import tpu_sc as plsc

S = 200
B = 4096
D = 64
DP = 128
L = 16
H = 64

CH = 8
NCHUNK = S // CH

_info = plsc.get_sparse_core_info()
NC = _info.num_cores
NS = _info.num_subcores
NW = NC * NS
BPW = B // NW


def _positional_encoding_np(seq_len: int, d_model: int) -> np.ndarray:
    position = np.arange(seq_len, dtype=np.float32)[:, None]
    div_term = np.exp(
        np.arange(0, d_model, 2, dtype=np.float32) * (-math.log(10000.0) / d_model)
    )
    pe = np.zeros((seq_len, d_model), dtype=np.float32)
    pe[:, 0::2] = np.sin(position * div_term)
    pe[:, 1::2] = np.cos(position * div_term)
    return pe


_PE = _positional_encoding_np(S, D)

_mesh = plsc.VectorSubcoreMesh(core_axis_name="c", subcore_axis_name="s")


@functools.partial(
    pl.kernel,
    mesh=_mesh,
    compiler_params=pltpu.CompilerParams(
        use_tc_tiling_on_sc=True, needs_layout_passes=False
    ),
    out_type=jax.ShapeDtypeStruct((2 * S, D, B), jnp.float32),
    scratch_types=[
        pltpu.VMEM((2, CH, BPW), jnp.int32),
        pltpu.VMEM((2, CH, BPW), jnp.int32),
        pltpu.VMEM((S * D + L,), jnp.float32),
        pltpu.VMEM((2, H, DP), jnp.float32),
        pltpu.VMEM((1000 * D,), jnp.float32),
        pltpu.VMEM((2, 2, D, BPW), jnp.float32),
        pltpu.SemaphoreType.DMA,
        pltpu.SemaphoreType.DMA,
        pltpu.SemaphoreType.DMA,
        pltpu.SemaphoreType.DMA,
        pltpu.SemaphoreType.DMA,
        pltpu.SemaphoreType.DMA,
        pltpu.SemaphoreType.DMA,
        pltpu.SemaphoreType.DMA,
    ],
)
def _embed_kernel(
    node_idx_hbm,
    edge_idx_hbm,
    node_table_hbm,
    edge_table_hbm,
    pe_hbm,
    out_hbm,
    idx_n,
    idx_e,
    pe_v,
    nbuf,
    etab,
    tbuf,
    gn0,
    gn1,
    sn0,
    sn1,
    se0,
    se1,
    pn_sem,
    pe_sem,
):
    cid = lax.axis_index("c")
    sid = lax.axis_index("s")
    wid = sid * NC + cid
    base = wid * BPW

    gn = (gn0, gn1)
    sn = (sn0, sn1)
    se = (se0, se1)

    pltpu.sync_copy(pe_hbm, pe_v.at[pl.ds(0, S * D)])
    pltpu.sync_copy(edge_table_hbm, etab)
    pltpu.sync_copy(node_idx_hbm.at[pl.ds(0, CH), pl.ds(base, BPW)], idx_n.at[0])
    pltpu.sync_copy(edge_idx_hbm.at[pl.ds(0, CH), pl.ds(base, BPW)], idx_e.at[0])

    def start_idx_prefetch(c1):
        kc = lax.rem(c1, 2)
        pltpu.async_copy(
            node_idx_hbm.at[pl.ds(c1 * CH, CH), pl.ds(base, BPW)], idx_n.at[kc], pn_sem
        )
        pltpu.async_copy(
            edge_idx_hbm.at[pl.ds(c1 * CH, CH), pl.ds(base, BPW)], idx_e.at[kc], pe_sem
        )

    def wait_idx_prefetch(c1):
        kc = lax.rem(c1, 2)
        pltpu.make_async_copy(
            node_idx_hbm.at[pl.ds(c1 * CH, CH), pl.ds(base, BPW)], idx_n.at[kc], pn_sem
        ).wait()
        pltpu.make_async_copy(
            edge_idx_hbm.at[pl.ds(c1 * CH, CH), pl.ds(base, BPW)], idx_e.at[kc], pe_sem
        ).wait()

    def start_gather(kc, j, h, k):
        pltpu.async_copy(
            node_table_hbm.at[idx_n.at[kc, j, pl.ds(h * H, H)]], nbuf.at[k], gn[k]
        )

    def wait_gather(kc, j, h, k):
        pltpu.make_async_copy(
            node_table_hbm.at[idx_n.at[kc, j, pl.ds(h * H, H)]], nbuf.at[k], gn[k]
        ).wait()

    def start_scatter(sp, kt):
        pltpu.async_copy(tbuf.at[kt, 0], out_hbm.at[sp, :, pl.ds(base, BPW)], sn[kt])
        pltpu.async_copy(
            tbuf.at[kt, 1], out_hbm.at[S + sp, :, pl.ds(base, BPW)], se[kt]
        )

    def wait_scatter(sp, kt):
        pltpu.make_async_copy(
            tbuf.at[kt, 0], out_hbm.at[sp, :, pl.ds(base, BPW)], sn[kt]
        ).wait()
        pltpu.make_async_copy(
            tbuf.at[kt, 1], out_hbm.at[S + sp, :, pl.ds(base, BPW)], se[kt]
        ).wait()

    nbuf2d = [nbuf.at[0], nbuf.at[1]]
    tb_n = [tbuf.at[0, 0], tbuf.at[1, 0]]
    tb_e = [tbuf.at[0, 1], tbuf.at[1, 1]]
    iota = lax.iota(jnp.int32, L)

    def compute_half(kc, j, sp, h, k, kt):
        src_n = nbuf2d[k]
        dst_n = tb_n[kt]
        dst_e = tb_e[kt]

        rowc = [iota + (t * L) for t in range(H // L)]
        colc = [iota + (h * H + t * L) for t in range(H // L)]
        peb = [jnp.full((L,), sp * D + q * L, jnp.int32) for q in range(D // L)]

        def sbody(sft, carry):
            perm = lax.rem(iota + sft, L)
            for q in range(D // L):
                dstrow = perm + (q * L)
                pe_p = plsc.load_gather(pe_v, [peb[q] + perm])
                for t in range(H // L):
                    v = plsc.load_gather(src_n, [rowc[t], dstrow])
                    plsc.store_scatter(dst_n, [dstrow, colc[t]], v + pe_p)
            return carry

        lax.fori_loop(0, L, sbody, None)

        eidxs = [idx_e[kc, j, pl.ds(h * H + t * L, L)] for t in range(H // L)]
        kthous = [jnp.full((L,), kk * 1000, jnp.int32) for kk in range(4)]

        def dbody(u, dbase):
            for uu in range(4):
                d = u * 4 + uu
                pe_s = jnp.full((L,), pe_v[pl.ds(sp * D + d, L)][0], jnp.float32)
                db = dbase + kthous[uu]
                for t in range(H // L):
                    ve = plsc.load_gather(etab, [db + eidxs[t]])
                    dst_e[d, pl.ds(h * H + t * L, L)] = ve + pe_s
            return dbase + 4000

        lax.fori_loop(0, D // 4, dbody, jnp.zeros((L,), jnp.int32))

    start_gather(0, 0, 0, 0)

    def chunk_body(c, carry):
        kc = lax.rem(c, 2)
        for j in range(CH):
            sp = c * CH + j
            kt = j % 2
            for h in range(2):
                k = h
                if h == 0:
                    start_gather(kc, j, 1, 1 - k)
                elif j == CH - 1:
                    @pl.when(c + 1 < NCHUNK)
                    def _():
                        wait_idx_prefetch(c + 1)
                        start_gather(lax.rem(c + 1, 2), 0, 0, 1 - k)

                else:
                    start_gather(kc, j + 1, 0, 1 - k)

                wait_gather(kc, j, h, k)

                if j == 0 and h == 0:
                    @pl.when(c + 1 < NCHUNK)
                    def _():
                        start_idx_prefetch(c + 1)

                if h == 0:
                    @pl.when(sp >= 2)
                    def _():
                        wait_scatter(sp - 2, kt)

                compute_half(kc, j, sp, h, k, kt)
            start_scatter(sp, kt)
        return carry

    lax.fori_loop(0, NCHUNK, chunk_body, None)

    wait_scatter(S - 2, 0)
    wait_scatter(S - 1, 1)


def kernel(node_idx, edge_idx, node_table, edge_table):
    pe = jnp.asarray(_PE.reshape(-1))
    node_pad = jnp.pad(node_table.astype(jnp.float32), ((0, 0), (0, DP - D)))
    out_t = _embed_kernel(
        node_idx.astype(jnp.int32),
        edge_idx.astype(jnp.int32),
        node_pad,
        edge_table.astype(jnp.float32).T.reshape(-1),
        pe,
    )
    return jnp.transpose(out_t, (0, 2, 1))


# speedup vs baseline: 1.6003x; 1.6003x over previous
import functools
import math

import jax
import jax.numpy as jnp
import numpy as np
from jax import lax
from jax.experimental import pallas as pl
from jax.experimental.pallas import tpu as pltpu
from jax.experimental.pallas import tpu_sc as plsc

S = 200
B = 4096
D = 64
DP = 128
L = 16
H = 64

CH = 8
NCHUNK = S // CH

_info = plsc.get_sparse_core_info()
NC = _info.num_cores
NS = _info.num_subcores
NW = NC * NS
BPW = B // NW


def _positional_encoding_np(seq_len: int, d_model: int) -> np.ndarray:
    position = np.arange(seq_len, dtype=np.float32)[:, None]
    div_term = np.exp(
        np.arange(0, d_model, 2, dtype=np.float32) * (-math.log(10000.0) / d_model)
    )
    pe = np.zeros((seq_len, d_model), dtype=np.float32)
    pe[:, 0::2] = np.sin(position * div_term)
    pe[:, 1::2] = np.cos(position * div_term)
    return pe


_PE = _positional_encoding_np(S, D)

_mesh = plsc.VectorSubcoreMesh(core_axis_name="c", subcore_axis_name="s")


@functools.partial(
    pl.kernel,
    mesh=_mesh,
    compiler_params=pltpu.CompilerParams(
        use_tc_tiling_on_sc=True, needs_layout_passes=False
    ),
    out_type=jax.ShapeDtypeStruct((2 * S, D, B), jnp.float32),
    scratch_types=[
        pltpu.VMEM((2, CH, BPW), jnp.int32),
        pltpu.VMEM((2, CH, BPW), jnp.int32),
        pltpu.VMEM((S * D + L,), jnp.float32),
        pltpu.VMEM((2, H, DP), jnp.float32),
        pltpu.VMEM((1000 * D,), jnp.float32),
        pltpu.VMEM((2, 2, D, BPW), jnp.float32),
        pltpu.SemaphoreType.DMA,
        pltpu.SemaphoreType.DMA,
        pltpu.SemaphoreType.DMA,
        pltpu.SemaphoreType.DMA,
        pltpu.SemaphoreType.DMA,
        pltpu.SemaphoreType.DMA,
        pltpu.SemaphoreType.DMA,
        pltpu.SemaphoreType.DMA,
    ],
)
def _embed_kernel(
    node_idx_hbm,
    edge_idx_hbm,
    node_table_hbm,
    edge_table_hbm,
    pe_hbm,
    out_hbm,
    idx_n,
    idx_e,
    pe_v,
    nbuf,
    etab,
    tbuf,
    gn0,
    gn1,
    sn0,
    sn1,
    se0,
    se1,
    pn_sem,
    pe_sem,
):
    cid = lax.axis_index("c")
    sid = lax.axis_index("s")
    wid = sid * NC + cid
    base = wid * BPW

    gn = (gn0, gn1)
    sn = (sn0, sn1)
    se = (se0, se1)

    pltpu.sync_copy(pe_hbm, pe_v.at[pl.ds(0, S * D)])
    pltpu.sync_copy(edge_table_hbm, etab)
    pltpu.sync_copy(node_idx_hbm.at[pl.ds(0, CH), pl.ds(base, BPW)], idx_n.at[0])
    pltpu.sync_copy(edge_idx_hbm.at[pl.ds(0, CH), pl.ds(base, BPW)], idx_e.at[0])

    def start_idx_prefetch(c1):
        kc = lax.rem(c1, 2)
        pltpu.async_copy(
            node_idx_hbm.at[pl.ds(c1 * CH, CH), pl.ds(base, BPW)], idx_n.at[kc], pn_sem
        )
        pltpu.async_copy(
            edge_idx_hbm.at[pl.ds(c1 * CH, CH), pl.ds(base, BPW)], idx_e.at[kc], pe_sem
        )

    def wait_idx_prefetch(c1):
        kc = lax.rem(c1, 2)
        pltpu.make_async_copy(
            node_idx_hbm.at[pl.ds(c1 * CH, CH), pl.ds(base, BPW)], idx_n.at[kc], pn_sem
        ).wait()
        pltpu.make_async_copy(
            edge_idx_hbm.at[pl.ds(c1 * CH, CH), pl.ds(base, BPW)], idx_e.at[kc], pe_sem
        ).wait()

    def start_gather(kc, j, h, k):
        pltpu.async_copy(
            node_table_hbm.at[idx_n.at[kc, j, pl.ds(h * H, H)]], nbuf.at[k], gn[k]
        )

    def wait_gather(kc, j, h, k):
        pltpu.make_async_copy(
            node_table_hbm.at[idx_n.at[kc, j, pl.ds(h * H, H)]], nbuf.at[k], gn[k]
        ).wait()

    def start_scatter(sp, kt):
        pltpu.async_copy(tbuf.at[kt, 0], out_hbm.at[sp, :, pl.ds(base, BPW)], sn[kt])
        pltpu.async_copy(
            tbuf.at[kt, 1], out_hbm.at[S + sp, :, pl.ds(base, BPW)], se[kt]
        )

    def wait_scatter(sp, kt):
        pltpu.make_async_copy(
            tbuf.at[kt, 0], out_hbm.at[sp, :, pl.ds(base, BPW)], sn[kt]
        ).wait()
        pltpu.make_async_copy(
            tbuf.at[kt, 1], out_hbm.at[S + sp, :, pl.ds(base, BPW)], se[kt]
        ).wait()

    nbuf2d = [nbuf.at[0], nbuf.at[1]]
    tb_n = [tbuf.at[0, 0], tbuf.at[1, 0]]
    tb_e = [tbuf.at[0, 1], tbuf.at[1, 1]]
    iota = lax.iota(jnp.int32, L)

    def compute_half(kc, j, sp, h, k, kt):
        src_n = nbuf2d[k]
        dst_n = tb_n[kt]
        dst_e = tb_e[kt]

        rowc = [iota + (t * L) for t in range(H // L)]
        colc = [iota + (h * H + t * L) for t in range(H // L)]
        peb = [jnp.full((L,), sp * D + q * L, jnp.int32) for q in range(D // L)]

        def sbody(sft, carry):
            perm = lax.rem(iota + sft, L)
            for q in range(D // L):
                dstrow = perm + (q * L)
                pe_p = plsc.load_gather(pe_v, [peb[q] + perm])
                for t in range(H // L):
                    v = plsc.load_gather(src_n, [rowc[t], dstrow])
                    plsc.store_scatter(dst_n, [dstrow, colc[t]], v + pe_p)
            return carry

        lax.fori_loop(0, L, sbody, None)

        eidxs = [idx_e[kc, j, pl.ds(h * H + t * L, L)] for t in range(H // L)]

        @plsc.parallel_loop(0, D, unroll=1)
        def dbody(d):
            pe_s = jnp.full((L,), pe_v[pl.ds(sp * D + d, L)][0], jnp.float32)
            db = jnp.full((L,), d * 1000, jnp.int32)
            for t in range(H // L):
                ve = plsc.load_gather(etab, [db + eidxs[t]])
                dst_e[d, pl.ds(h * H + t * L, L)] = ve + pe_s

    start_gather(0, 0, 0, 0)

    def chunk_body(c, carry):
        kc = lax.rem(c, 2)
        for j in range(CH):
            sp = c * CH + j
            kt = j % 2
            for h in range(2):
                k = h
                if h == 0:
                    start_gather(kc, j, 1, 1 - k)
                elif j == CH - 1:
                    @pl.when(c + 1 < NCHUNK)
                    def _():
                        wait_idx_prefetch(c + 1)
                        start_gather(lax.rem(c + 1, 2), 0, 0, 1 - k)

                else:
                    start_gather(kc, j + 1, 0, 1 - k)

                wait_gather(kc, j, h, k)

                if j == 0 and h == 0:
                    @pl.when(c + 1 < NCHUNK)
                    def _():
                        start_idx_prefetch(c + 1)

                if h == 0:
                    @pl.when(sp >= 2)
                    def _():
                        wait_scatter(sp - 2, kt)

                compute_half(kc, j, sp, h, k, kt)
            start_scatter(sp, kt)
        return carry

    lax.fori_loop(0, NCHUNK, chunk_body, None)

    wait_scatter(S - 2, 0)
    wait_scatter(S - 1, 1)


def kernel(node_idx, edge_idx, node_table, edge_table):
    pe = jnp.asarray(_PE.reshape(-1))
    node_pad = jnp.pad(node_table.astype(jnp.float32), ((0, 0), (0, DP - D)))
    out_t = _embed_kernel(
        node_idx.astype(jnp.int32),
        edge_idx.astype(jnp.int32),
        node_pad,
        edge_table.astype(jnp.float32).T.reshape(-1),
        pe,
    )
    return jnp.transpose(out_t, (0, 2, 1))
